# Initial kernel scaffold; baseline (speedup 1.0000x reference)
import jax
import jax.numpy as jnp
from jax.experimental import pallas as pl


def kernel(x, edge_index, W1, b1, W2, b2, comp1, comp2):
    raise NotImplementedError("write your pallas kernel here")


# speedup vs baseline: 12.0361x; 12.0361x over previous
import functools

import jax
import jax.numpy as jnp
from jax import lax
from jax.experimental import pallas as pl
from jax.experimental.pallas import tpu as pltpu
from jax.experimental.pallas import tpu_sc as plsc

NC = 2
NS = 16
NW = NC * NS

EB = 80
HW = 128

_sc_mesh = plsc.VectorSubcoreMesh(core_axis_name="c", subcore_axis_name="s",
                                  num_cores=NC, num_subcores=NS)


def _make_hist(n, e):
    chunk = e // NW
    nb = chunk // EB

    @functools.partial(
        pl.kernel,
        out_type=jax.ShapeDtypeStruct((NC, n, HW), jnp.float32),
        mesh=_sc_mesh,
        scratch_types=[
            pltpu.VMEM((EB,), jnp.int32),
            pltpu.VMEM((EB, HW), jnp.float32),
            pltpu.VMEM_SHARED((n, HW), jnp.float32),
        ],
    )
    def hist(dst_hbm, zeros_hbm, ones_hbm, out_hbm, idx_v, ones_v, hist_sh):
        c = lax.axis_index("c")
        s = lax.axis_index("s")
        wid = s * NC + c

        @pl.when(s == 0)
        def _():
            pltpu.sync_copy(zeros_hbm, hist_sh)

        pltpu.sync_copy(ones_hbm, ones_v)
        plsc.subcore_barrier()

        def body(j, carry):
            base = wid * chunk + j * EB
            pltpu.sync_copy(dst_hbm.at[pl.ds(base, EB)], idx_v)
            pltpu.sync_copy(ones_v, hist_sh.at[idx_v], add=True)
            return carry

        lax.fori_loop(0, nb, body, 0)
        plsc.subcore_barrier()

        @pl.when(s == 0)
        def _():
            pltpu.sync_copy(hist_sh, out_hbm.at[c])

    return hist


def _make_edge_agg(n, e, d):
    chunk = e // NW
    nb = chunk // EB

    @functools.partial(
        pl.kernel,
        out_type=jax.ShapeDtypeStruct((NC, n, d), jnp.float32),
        mesh=_sc_mesh,
        scratch_types=[
            pltpu.VMEM((EB,), jnp.int32),
            pltpu.VMEM((EB,), jnp.int32),
            pltpu.VMEM((EB, d), jnp.float32),
            pltpu.VMEM_SHARED((n, d), jnp.float32),
            pltpu.SemaphoreType.DMA,
        ],
    )
    def edge_agg(y_hbm, src_hbm, dst_hbm, zeros_hbm, out_hbm,
                 src_v, dst_v, rows_v, acc_sh, sem):
        c = lax.axis_index("c")
        s = lax.axis_index("s")
        wid = s * NC + c

        @pl.when(s == 0)
        def _():
            pltpu.sync_copy(zeros_hbm, acc_sh)

        plsc.subcore_barrier()

        def body(j, carry):
            base = wid * chunk + j * EB
            pltpu.sync_copy(src_hbm.at[pl.ds(base, EB)], src_v)
            pltpu.sync_copy(dst_hbm.at[pl.ds(base, EB)], dst_v)
            pltpu.async_copy(y_hbm.at[src_v], rows_v, sem).wait()
            pltpu.sync_copy(rows_v, acc_sh.at[dst_v], add=True)
            return carry

        lax.fori_loop(0, nb, body, 0)
        plsc.subcore_barrier()

        @pl.when(s == 0)
        def _():
            pltpu.sync_copy(acc_sh, out_hbm.at[c])

    return edge_agg


def _dinv_block(h0, h1):
    deg = h0[:, 0:1] + h1[:, 0:1] + 1.0
    return lax.rsqrt(deg)


def _tc_first(x_ref, w_ref, h0_ref, h1_ref, y_ref):
    dinv = _dinv_block(h0_ref[...], h1_ref[...])
    xw = jnp.dot(x_ref[...], w_ref[...], preferred_element_type=jnp.float32)
    y_ref[...] = xw * dinv


def _tc_mid(a0_ref, a1_ref, y_ref, h0_ref, h1_ref, b_ref, comp_ref, w_ref,
            out_ref):
    dinv = _dinv_block(h0_ref[...], h1_ref[...])
    conv = dinv * (a0_ref[...] + a1_ref[...] + y_ref[...]) + b_ref[...]
    h = jnp.maximum(conv + comp_ref[...], 0.0)
    hw = jnp.dot(h, w_ref[...], preferred_element_type=jnp.float32)
    out_ref[...] = hw * dinv


def _tc_last(a0_ref, a1_ref, y_ref, h0_ref, h1_ref, b_ref, comp_ref, out_ref):
    dinv = _dinv_block(h0_ref[...], h1_ref[...])
    out_ref[...] = (dinv * (a0_ref[...] + a1_ref[...] + y_ref[...])
                    + b_ref[...] + comp_ref[...])


def kernel(x, edge_index, W1, b1, W2, b2, comp1, comp2):
    n, d = x.shape
    e = edge_index.shape[1]
    assert e % NW == 0 and (e // NW) % EB == 0

    src = edge_index[0]
    dst = edge_index[1]

    hist_fn = _make_hist(n, e)
    agg_fn = _make_edge_agg(n, e, d)

    zeros_h = jnp.zeros((n, HW), jnp.float32)
    ones_h = jnp.ones((EB, HW), jnp.float32)
    zeros_a = jnp.zeros((n, d), jnp.float32)

    hist = hist_fn(dst, zeros_h, ones_h)
    h0, h1 = hist[0], hist[1]

    r = 1000
    grid = (n // r,)
    row_spec = pl.BlockSpec((r, d), lambda i: (i, 0))
    hist_spec = pl.BlockSpec((r, HW), lambda i: (i, 0))
    w_spec = pl.BlockSpec((d, d), lambda i: (0, 0))
    b_spec = pl.BlockSpec((1, d), lambda i: (0, 0))

    b1r = b1.reshape(1, d)
    b2r = b2.reshape(1, d)

    y1 = pl.pallas_call(
        _tc_first,
        grid=grid,
        in_specs=[row_spec, w_spec, hist_spec, hist_spec],
        out_specs=row_spec,
        out_shape=jax.ShapeDtypeStruct((n, d), jnp.float32),
    )(x, W1, h0, h1)

    acc1 = agg_fn(y1, src, dst, zeros_a)

    y2 = pl.pallas_call(
        _tc_mid,
        grid=grid,
        in_specs=[row_spec, row_spec, row_spec, hist_spec, hist_spec,
                  b_spec, row_spec, w_spec],
        out_specs=row_spec,
        out_shape=jax.ShapeDtypeStruct((n, d), jnp.float32),
    )(acc1[0], acc1[1], y1, h0, h1, b1r, comp1, W2)

    acc2 = agg_fn(y2, src, dst, zeros_a)

    out = pl.pallas_call(
        _tc_last,
        grid=grid,
        in_specs=[row_spec, row_spec, row_spec, hist_spec, hist_spec,
                  b_spec, row_spec],
        out_specs=row_spec,
        out_shape=jax.ShapeDtypeStruct((n, d), jnp.float32),
    )(acc2[0], acc2[1], y2, h0, h1, b2r, comp2)

    return out
